# baseline (speedup 1.0000x reference)
import jax
import jax.numpy as jnp
from jax import lax
from jax.experimental import pallas as pl

_OUTPUTS = 32768
_ROWS_PER_STEP = 8

_K1 = (1832780943, 270669613)
_K2 = (64467757, 2916123636)

_TINY = 1.1754943508222875e-38


def _rotl(x, d):
    return (x << jnp.uint32(d)) | (x >> jnp.uint32(32 - d))


def _threefry_bits(key, x1):
    k1, k2 = key
    ks = (jnp.uint32(k1), jnp.uint32(k2), jnp.uint32(k1 ^ k2 ^ 0x1BD11BDA))
    rots = ((13, 15, 26, 6), (17, 29, 16, 24))
    x0 = jnp.full_like(x1, ks[0])
    x1 = x1 + ks[1]
    for i in range(5):
        for r in rots[i % 2]:
            x0 = x0 + x1
            x1 = _rotl(x1, r) ^ x0
        x0 = x0 + ks[(i + 1) % 3]
        x1 = x1 + ks[(i + 2) % 3] + jnp.uint32(i + 1)
    return x0 ^ x1


def _bits_to_unit(bits):
    f = lax.bitcast_convert_type(
        (bits >> jnp.uint32(9)) | jnp.uint32(0x3F800000), jnp.float32)
    return f - jnp.float32(1.0)


_CHUNK = 2048


def _sample_body(o_ref):
    pid = pl.program_id(0)
    shape = (_ROWS_PER_STEP, _CHUNK)
    nch = _OUTPUTS // _CHUNK
    base = (pid * (_ROWS_PER_STEP * _OUTPUTS)).astype(jnp.uint32)
    rowoff = lax.broadcasted_iota(jnp.uint32, shape, 0) * jnp.uint32(_OUTPUTS)
    lane = lax.broadcasted_iota(jnp.uint32, shape, 1)
    idx0 = base + rowoff + lane
    lanei = lax.broadcasted_iota(jnp.int32, shape, 1)

    def chunk(c, carry):
        vmin, vidx = carry
        idx = idx0 + (c * _CHUNK).astype(jnp.uint32)
        u1 = _bits_to_unit(_threefry_bits(_K1, idx))
        f2 = _bits_to_unit(_threefry_bits(_K2, idx))
        tiny = jnp.float32(_TINY)
        u2 = jnp.maximum(tiny, f2 + tiny)
        r = -jnp.log(u2) / u1
        coli = lanei + c * _CHUNK
        m = r < vmin
        return jnp.where(m, r, vmin), jnp.where(m, coli, vidx)

    vmin0 = jnp.full(shape, jnp.inf, jnp.float32)
    vidx0 = jnp.zeros(shape, jnp.int32)
    vmin, vidx = lax.fori_loop(0, nch, chunk, (vmin0, vidx0))

    rmin = jnp.min(vmin, axis=1, keepdims=True)
    cand = jnp.where(vmin == rmin, vidx, jnp.int32(2**31 - 1))
    winners = jnp.min(cand, axis=1)
    o_ref[pl.ds(pid, 1), :] = winners.reshape(1, _ROWS_PER_STEP)


def kernel(x):
    batch = x.shape[0]
    steps = batch // _ROWS_PER_STEP
    out = pl.pallas_call(
        _sample_body,
        grid=(steps,),
        out_specs=pl.BlockSpec((steps, _ROWS_PER_STEP), lambda i: (0, 0)),
        out_shape=jax.ShapeDtypeStruct((steps, _ROWS_PER_STEP), jnp.int32),
    )()
    return out.reshape(batch)


# speedup vs baseline: 1.0445x; 1.0445x over previous
import jax
import jax.numpy as jnp
from jax import lax
from jax.experimental import pallas as pl

_OUTPUTS = 32768
_ROWS_PER_STEP = 8

_K1 = (1832780943, 270669613)
_K2 = (64467757, 2916123636)

_TINY = 1.1754943508222875e-38


def _rotl(x, d):
    return (x << jnp.uint32(d)) | (x >> jnp.uint32(32 - d))


def _threefry_bits(key, x1):
    k1, k2 = key
    ks = (jnp.uint32(k1), jnp.uint32(k2), jnp.uint32(k1 ^ k2 ^ 0x1BD11BDA))
    rots = ((13, 15, 26, 6), (17, 29, 16, 24))
    x0 = jnp.full_like(x1, ks[0])
    x1 = x1 + ks[1]
    for i in range(5):
        for r in rots[i % 2]:
            x0 = x0 + x1
            x1 = _rotl(x1, r) ^ x0
        x0 = x0 + ks[(i + 1) % 3]
        x1 = x1 + ks[(i + 2) % 3] + jnp.uint32(i + 1)
    return x0 ^ x1


def _bits_to_unit(bits):
    f = lax.bitcast_convert_type(
        (bits >> jnp.uint32(9)) | jnp.uint32(0x3F800000), jnp.float32)
    return f - jnp.float32(1.0)


_CHUNK = 1024


def _sample_body(o_ref):
    pid = pl.program_id(0)
    shape = (_ROWS_PER_STEP, _CHUNK)
    nch = _OUTPUTS // _CHUNK
    base = (pid * (_ROWS_PER_STEP * _OUTPUTS)).astype(jnp.uint32)
    rowoff = lax.broadcasted_iota(jnp.uint32, shape, 0) * jnp.uint32(_OUTPUTS)
    lane = lax.broadcasted_iota(jnp.uint32, shape, 1)
    idx0 = base + rowoff + lane
    lanei = lax.broadcasted_iota(jnp.int32, shape, 1)

    def chunk(c, carry):
        vmin, vidx = carry
        idx = idx0 + (c * _CHUNK).astype(jnp.uint32)
        u1 = _bits_to_unit(_threefry_bits(_K1, idx))
        f2 = _bits_to_unit(_threefry_bits(_K2, idx))
        tiny = jnp.float32(_TINY)
        u2 = jnp.maximum(tiny, f2 + tiny)
        r = -jnp.log(u2) / u1
        coli = lanei + c * _CHUNK
        m = r < vmin
        return jnp.where(m, r, vmin), jnp.where(m, coli, vidx)

    vmin0 = jnp.full(shape, jnp.inf, jnp.float32)
    vidx0 = jnp.zeros(shape, jnp.int32)
    vmin, vidx = lax.fori_loop(0, nch, chunk, (vmin0, vidx0), unroll=2)

    rmin = jnp.min(vmin, axis=1, keepdims=True)
    cand = jnp.where(vmin == rmin, vidx, jnp.int32(2**31 - 1))
    winners = jnp.min(cand, axis=1)
    o_ref[pl.ds(pid, 1), :] = winners.reshape(1, _ROWS_PER_STEP)


def kernel(x):
    batch = x.shape[0]
    steps = batch // _ROWS_PER_STEP
    out = pl.pallas_call(
        _sample_body,
        grid=(steps,),
        out_specs=pl.BlockSpec((steps, _ROWS_PER_STEP), lambda i: (0, 0)),
        out_shape=jax.ShapeDtypeStruct((steps, _ROWS_PER_STEP), jnp.int32),
    )()
    return out.reshape(batch)


# speedup vs baseline: 1.0542x; 1.0092x over previous
import jax
import jax.numpy as jnp
from jax import lax
from jax.experimental import pallas as pl

_OUTPUTS = 32768
_ROWS_PER_STEP = 8

_K1 = (1832780943, 270669613)
_K2 = (64467757, 2916123636)

_TINY = 1.1754943508222875e-38


def _rotl(x, d):
    return (x << jnp.uint32(d)) | (x >> jnp.uint32(32 - d))


def _threefry_bits(key, x1):
    k1, k2 = key
    ks = (jnp.uint32(k1), jnp.uint32(k2), jnp.uint32(k1 ^ k2 ^ 0x1BD11BDA))
    rots = ((13, 15, 26, 6), (17, 29, 16, 24))
    x0 = jnp.full_like(x1, ks[0])
    x1 = x1 + ks[1]
    for i in range(5):
        for r in rots[i % 2]:
            x0 = x0 + x1
            x1 = _rotl(x1, r) ^ x0
        x0 = x0 + ks[(i + 1) % 3]
        x1 = x1 + ks[(i + 2) % 3] + jnp.uint32(i + 1)
    return x0 ^ x1


def _bits_to_unit(bits):
    f = lax.bitcast_convert_type(
        (bits >> jnp.uint32(9)) | jnp.uint32(0x3F800000), jnp.float32)
    return f - jnp.float32(1.0)


_CHUNK = 1024


def _sample_body(o_ref):
    pid = pl.program_id(0)
    shape = (_ROWS_PER_STEP, _CHUNK)
    nch = _OUTPUTS // _CHUNK
    base = (pid * (_ROWS_PER_STEP * _OUTPUTS)).astype(jnp.uint32)
    rowoff = lax.broadcasted_iota(jnp.uint32, shape, 0) * jnp.uint32(_OUTPUTS)
    lane = lax.broadcasted_iota(jnp.uint32, shape, 1)
    idx0 = base + rowoff + lane
    lanei = lax.broadcasted_iota(jnp.int32, shape, 1)

    def chunk(c, carry):
        vmin, vidx = carry
        idx = idx0 + (c * _CHUNK).astype(jnp.uint32)
        u1 = _bits_to_unit(_threefry_bits(_K1, idx))
        f2 = _bits_to_unit(_threefry_bits(_K2, idx))
        tiny = jnp.float32(_TINY)
        u2 = jnp.maximum(tiny, f2 + tiny)
        r = -jnp.log(u2) / u1
        coli = lanei + c * _CHUNK
        m = r < vmin
        return jnp.where(m, r, vmin), jnp.where(m, coli, vidx)

    vmin0 = jnp.full(shape, jnp.inf, jnp.float32)
    vidx0 = jnp.zeros(shape, jnp.int32)
    vmin, vidx = lax.fori_loop(0, nch, chunk, (vmin0, vidx0), unroll=4)

    rmin = jnp.min(vmin, axis=1, keepdims=True)
    cand = jnp.where(vmin == rmin, vidx, jnp.int32(2**31 - 1))
    winners = jnp.min(cand, axis=1)
    o_ref[pl.ds(pid, 1), :] = winners.reshape(1, _ROWS_PER_STEP)


def kernel(x):
    batch = x.shape[0]
    steps = batch // _ROWS_PER_STEP
    out = pl.pallas_call(
        _sample_body,
        grid=(steps,),
        out_specs=pl.BlockSpec((steps, _ROWS_PER_STEP), lambda i: (0, 0)),
        out_shape=jax.ShapeDtypeStruct((steps, _ROWS_PER_STEP), jnp.int32),
    )()
    return out.reshape(batch)


# speedup vs baseline: 1.0558x; 1.0016x over previous
import jax
import jax.numpy as jnp
from jax import lax
from jax.experimental import pallas as pl

_OUTPUTS = 32768
_ROWS_PER_STEP = 8

_K1 = (1832780943, 270669613)
_K2 = (64467757, 2916123636)

_TINY = 1.1754943508222875e-38


def _rotl(x, d):
    return (x << jnp.uint32(d)) | (x >> jnp.uint32(32 - d))


def _threefry_bits(key, x1):
    k1, k2 = key
    ks = (jnp.uint32(k1), jnp.uint32(k2), jnp.uint32(k1 ^ k2 ^ 0x1BD11BDA))
    rots = ((13, 15, 26, 6), (17, 29, 16, 24))
    x0 = jnp.full_like(x1, ks[0])
    x1 = x1 + ks[1]
    for i in range(5):
        for r in rots[i % 2]:
            x0 = x0 + x1
            x1 = _rotl(x1, r) ^ x0
        x0 = x0 + ks[(i + 1) % 3]
        x1 = x1 + ks[(i + 2) % 3] + jnp.uint32(i + 1)
    return x0 ^ x1


def _bits_to_unit(bits):
    f = lax.bitcast_convert_type(
        (bits >> jnp.uint32(9)) | jnp.uint32(0x3F800000), jnp.float32)
    return f - jnp.float32(1.0)


_CHUNK = 1024


def _sample_body(o_ref):
    pid = pl.program_id(0)
    shape = (_ROWS_PER_STEP, _CHUNK)
    nch = _OUTPUTS // _CHUNK
    base = (pid * (_ROWS_PER_STEP * _OUTPUTS)).astype(jnp.uint32)
    rowoff = lax.broadcasted_iota(jnp.uint32, shape, 0) * jnp.uint32(_OUTPUTS)
    lane = lax.broadcasted_iota(jnp.uint32, shape, 1)
    idx0 = base + rowoff + lane
    lanei = lax.broadcasted_iota(jnp.int32, shape, 1)

    def chunk(c, carry):
        vmin, vidx = carry
        idx = idx0 + (c * _CHUNK).astype(jnp.uint32)
        u1 = _bits_to_unit(_threefry_bits(_K1, idx))
        f2 = _bits_to_unit(_threefry_bits(_K2, idx))
        tiny = jnp.float32(_TINY)
        u2 = jnp.maximum(tiny, f2 + tiny)
        r = -jnp.log(u2) / u1
        coli = lanei + c * _CHUNK
        m = r < vmin
        return jnp.where(m, r, vmin), jnp.where(m, coli, vidx)

    vmin0 = jnp.full(shape, jnp.inf, jnp.float32)
    vidx0 = jnp.zeros(shape, jnp.int32)
    vmin, vidx = lax.fori_loop(0, nch, chunk, (vmin0, vidx0), unroll=8)

    rmin = jnp.min(vmin, axis=1, keepdims=True)
    cand = jnp.where(vmin == rmin, vidx, jnp.int32(2**31 - 1))
    winners = jnp.min(cand, axis=1)
    o_ref[pl.ds(pid, 1), :] = winners.reshape(1, _ROWS_PER_STEP)


def kernel(x):
    batch = x.shape[0]
    steps = batch // _ROWS_PER_STEP
    out = pl.pallas_call(
        _sample_body,
        grid=(steps,),
        out_specs=pl.BlockSpec((steps, _ROWS_PER_STEP), lambda i: (0, 0)),
        out_shape=jax.ShapeDtypeStruct((steps, _ROWS_PER_STEP), jnp.int32),
    )()
    return out.reshape(batch)
